# baseline (speedup 1.0000x reference)
import jax
import jax.numpy as jnp
from jax import lax
from jax.experimental import pallas as pl
from jax.experimental.pallas import tpu as pltpu
from jax.experimental.pallas import tpu_sc as plsc

_B = 16384
_F = 64
_NC = 2
_NS = 16
_L = 16
_NW = _NC * _NS
_BPW = _B // _NW
_NGRP = _BPW // _L
_BLK = 2048


def _sc_body(idx_hbm, pop_hbm, out_hbm, idx_v, pops_v, out_v, sem, gsem):
    wid = lax.axis_index("s") * _NC + lax.axis_index("c")
    base = wid * _BPW
    pltpu.sync_copy(idx_hbm.at[pl.ds(base, _BPW)], idx_v)
    c_pop = pltpu.async_copy(pop_hbm.at[idx_v], pops_v, gsem)
    c_pop.wait()

    def grp_body(g, carry):
        pops = pops_v[pl.ds(g * _L, _L)]
        bits = plsc.bitcast(pops, jnp.int32)
        y = plsc.bitcast(jnp.int32(0x5F3759DF) - (bits >> 1), jnp.float32)
        y = y * (1.5 - 0.5 * pops * y * y)
        y = y * (1.5 - 0.5 * pops * y * y)
        out_v[pl.ds(g * _L, _L)] = pops * y
        return carry

    lax.fori_loop(0, _NGRP, grp_body, 0)
    pltpu.sync_copy(out_v, out_hbm.at[pl.ds(base, _BPW)])


_sc_gather_sqrt = pl.kernel(
    _sc_body,
    out_type=jax.ShapeDtypeStruct((_B,), jnp.float32),
    mesh=plsc.VectorSubcoreMesh(core_axis_name="c", subcore_axis_name="s"),
    compiler_params=pltpu.CompilerParams(needs_layout_passes=False),
    scratch_types=[
        pltpu.VMEM((_BPW,), jnp.int32),
        pltpu.VMEM((_BPW,), jnp.float32),
        pltpu.VMEM((_BPW,), jnp.float32),
        pltpu.SemaphoreType.DMA,
        pltpu.SemaphoreType.DMA,
    ],
)


def _tc_dense_body(ut_ref, it_ref, wu_ref, wi_ref, out_ref):
    u = ut_ref[...]
    i = it_ref[...]
    wu = wu_ref[...]
    wi = wi_ref[...]
    du = jnp.sum(u * wu, axis=0)
    di = jnp.sum(i * wi, axis=0)
    den = (1.0 + jnp.exp(-du)) * (1.0 + jnp.exp(-di))
    out_ref[...] = 1.0 / den + 1.0


_tc_dense = pl.pallas_call(
    _tc_dense_body,
    out_shape=jax.ShapeDtypeStruct((_B,), jnp.float32),
    grid=(_B // _BLK,),
    in_specs=[
        pl.BlockSpec((_F, _BLK), lambda b: (0, b)),
        pl.BlockSpec((_F, _BLK), lambda b: (0, b)),
        pl.BlockSpec((_F, 1), lambda b: (0, 0)),
        pl.BlockSpec((_F, 1), lambda b: (0, 0)),
    ],
    out_specs=pl.BlockSpec((_BLK,), lambda b: (b,)),
)


def _tc_comb_body(s_ref, q_ref, ic_ref, out_ref):
    out_ref[...] = s_ref[...] * q_ref[...] + ic_ref[0]


_tc_comb = pl.pallas_call(
    _tc_comb_body,
    out_shape=jax.ShapeDtypeStruct((_B,), jnp.float32),
    grid=(_B // _BLK,),
    in_specs=[
        pl.BlockSpec((_BLK,), lambda b: (b,)),
        pl.BlockSpec((_BLK,), lambda b: (b,)),
        pl.BlockSpec(memory_space=pltpu.SMEM),
    ],
    out_specs=pl.BlockSpec((_BLK,), lambda b: (b,)),
)


@jax.jit
def kernel(users, items, item_pop_idx, W_user, W_item, intercept, popularity):
    idx = item_pop_idx.astype(jnp.int32)
    sqrt_pops = _sc_gather_sqrt(idx, popularity)
    scores = _tc_dense(users.T, items.T, W_user.astype(jnp.float32),
                       W_item.astype(jnp.float32))
    icpt = intercept.astype(jnp.float32).reshape(1)
    return _tc_comb(scores, sqrt_pops, icpt)


# speedup vs baseline: 1.2479x; 1.2479x over previous
import jax
import jax.numpy as jnp
from jax import lax
from jax.experimental import pallas as pl
from jax.experimental.pallas import tpu as pltpu
from jax.experimental.pallas import tpu_sc as plsc

_B = 16384
_F = 64
_NC = 2
_NS = 16
_L = 16
_NW = _NC * _NS
_BPW = _B // _NW
_NGRP = _BPW // _L
_BLK = 4096


def _sc_body(idx_hbm, pop_hbm, out_hbm, idx_v, pops_v, out_v, sem, gsem):
    wid = lax.axis_index("s") * _NC + lax.axis_index("c")
    base = wid * _BPW
    pltpu.sync_copy(idx_hbm.at[pl.ds(base, _BPW)], idx_v)
    c_pop = pltpu.async_copy(pop_hbm.at[idx_v], pops_v, gsem)
    c_pop.wait()

    def grp_body(g, carry):
        pops = pops_v[pl.ds(g * _L, _L)]
        bits = plsc.bitcast(pops, jnp.int32)
        y = plsc.bitcast(jnp.int32(0x5F3759DF) - (bits >> 1), jnp.float32)
        y = y * (1.5 - 0.5 * pops * y * y)
        y = y * (1.5 - 0.5 * pops * y * y)
        out_v[pl.ds(g * _L, _L)] = pops * y
        return carry

    lax.fori_loop(0, _NGRP, grp_body, 0)
    pltpu.sync_copy(out_v, out_hbm.at[pl.ds(base, _BPW)])


_sc_gather_sqrt = pl.kernel(
    _sc_body,
    out_type=jax.ShapeDtypeStruct((_B,), jnp.float32),
    mesh=plsc.VectorSubcoreMesh(core_axis_name="c", subcore_axis_name="s"),
    compiler_params=pltpu.CompilerParams(needs_layout_passes=False),
    scratch_types=[
        pltpu.VMEM((_BPW,), jnp.int32),
        pltpu.VMEM((_BPW,), jnp.float32),
        pltpu.VMEM((_BPW,), jnp.float32),
        pltpu.SemaphoreType.DMA,
        pltpu.SemaphoreType.DMA,
    ],
)


def _tc_dense_body(ut_ref, it_ref, wu_ref, wi_ref, out_ref):
    u = ut_ref[...]
    i = it_ref[...]
    wu = wu_ref[...]
    wi = wi_ref[...]
    du = jnp.dot(wu, u, preferred_element_type=jnp.float32)
    di = jnp.dot(wi, i, preferred_element_type=jnp.float32)
    den = (1.0 + jnp.exp(-du)) * (1.0 + jnp.exp(-di))
    out_ref[...] = 1.0 / den + 1.0


_tc_dense = pl.pallas_call(
    _tc_dense_body,
    out_shape=jax.ShapeDtypeStruct((1, _B), jnp.float32),
    grid=(_B // _BLK,),
    in_specs=[
        pl.BlockSpec((_F, _BLK), lambda b: (0, b)),
        pl.BlockSpec((_F, _BLK), lambda b: (0, b)),
        pl.BlockSpec((1, _F), lambda b: (0, 0)),
        pl.BlockSpec((1, _F), lambda b: (0, 0)),
    ],
    out_specs=pl.BlockSpec((1, _BLK), lambda b: (0, b)),
)


def _tc_comb_body(s_ref, q_ref, ic_ref, out_ref):
    out_ref[...] = s_ref[...] * q_ref[...] + ic_ref[0]


_tc_comb = pl.pallas_call(
    _tc_comb_body,
    out_shape=jax.ShapeDtypeStruct((_B // 128, 128), jnp.float32),
    in_specs=[
        pl.BlockSpec((_B // 128, 128), lambda: (0, 0)),
        pl.BlockSpec((_B // 128, 128), lambda: (0, 0)),
        pl.BlockSpec(memory_space=pltpu.SMEM),
    ],
    out_specs=pl.BlockSpec((_B // 128, 128), lambda: (0, 0)),
)


@jax.jit
def kernel(users, items, item_pop_idx, W_user, W_item, intercept, popularity):
    idx = item_pop_idx.astype(jnp.int32)
    sqrt_pops = _sc_gather_sqrt(idx, popularity)
    scores = _tc_dense(users.T, items.T,
                       W_user.astype(jnp.float32).T,
                       W_item.astype(jnp.float32).T)
    icpt = intercept.astype(jnp.float32).reshape(1)
    out2d = _tc_comb(scores.reshape(_B // 128, 128),
                     sqrt_pops.reshape(_B // 128, 128), icpt)
    return out2d.reshape(_B)
